# baseline (speedup 1.0000x reference)
import functools

import jax
import jax.numpy as jnp
from jax import lax
from jax.experimental import pallas as pl
from jax.experimental.pallas import tpu as pltpu
from jax.experimental.pallas import tpu_sc as plsc

N = 100000
E = 6400000
H = 128

NP = 102400

NUM_CORES = 2
NUM_SUBCORES = 16
NUM_WORKERS = NUM_CORES * NUM_SUBCORES
EDGES_PER_WORKER = E // NUM_WORKERS
CHUNK = 8000
NUM_CHUNKS = EDGES_PER_WORKER // CHUNK
SLOTS = 4
PIECE = NP // NUM_SUBCORES

CB = 6400
GRID = NP // CB


def _sc_segment_sums(x_flat, zeros_np, ones_chunk, ei_flat):
    mesh = plsc.VectorSubcoreMesh(core_axis_name="c", subcore_axis_name="s")

    @functools.partial(
        pl.kernel,
        out_type=jax.ShapeDtypeStruct((4 * NP,), jnp.float32),
        mesh=mesh,
        scratch_types=(
            [pltpu.VMEM((CHUNK,), jnp.int32)] * SLOTS
            + [pltpu.VMEM((CHUNK,), jnp.int32)] * SLOTS
            + [pltpu.VMEM((CHUNK,), jnp.float32)] * SLOTS
            + [
                pltpu.VMEM((CHUNK,), jnp.float32),
                pltpu.VMEM((PIECE,), jnp.float32),
                pltpu.VMEM_SHARED((NP,), jnp.float32),
                pltpu.VMEM_SHARED((NP,), jnp.float32),
                pltpu.VMEM_SHARED((NP,), jnp.float32),
            ]
            + [pltpu.SemaphoreType.DMA] * (3 * SLOTS)
        ),
    )
    def sc_kernel(x_hbm, zero_hbm, ones_hbm, ei_hbm, out_hbm, *scratch):
        src_v = scratch[0:SLOTS]
        dst_v = scratch[SLOTS:2 * SLOTS]
        vals_v = scratch[2 * SLOTS:3 * SLOTS]
        ones_v, piece_v, x_sp, sum_sp, cnt_sp = scratch[3 * SLOTS:3 * SLOTS + 5]
        sems = scratch[3 * SLOTS + 5:]
        sem_i = sems[0:SLOTS]
        sem_g = sems[SLOTS:2 * SLOTS]
        sem_s = sems[2 * SLOTS:3 * SLOTS]
        cid = lax.axis_index("c")
        sid = lax.axis_index("s")
        off = sid * PIECE

        pltpu.sync_copy(x_hbm.at[pl.ds(off, PIECE)], piece_v)
        pltpu.sync_copy(piece_v, x_sp.at[pl.ds(off, PIECE)])
        pltpu.sync_copy(zero_hbm.at[pl.ds(off, PIECE)], piece_v)
        pltpu.sync_copy(piece_v, sum_sp.at[pl.ds(off, PIECE)])
        pltpu.sync_copy(piece_v, cnt_sp.at[pl.ds(off, PIECE)])
        pltpu.sync_copy(ones_hbm, ones_v)

        plsc.subcore_barrier()

        wid = cid * NUM_SUBCORES + sid
        ebase = wid * EDGES_PER_WORKER

        def start_loads(j):
            b = j % SLOTS
            base = ebase + j * CHUNK
            d1 = pltpu.async_copy(
                ei_hbm.at[pl.ds(base, CHUNK)], src_v[b], sem_i[b])
            d2 = pltpu.async_copy(
                ei_hbm.at[pl.ds(E + base, CHUNK)], dst_v[b], sem_i[b])
            return (d1, d2)

        idx_descs = {0: start_loads(0), 1: start_loads(1)}
        scat_descs = {}
        for j in range(NUM_CHUNKS):
            b = j % SLOTS
            for d in idx_descs.pop(j):
                d.wait()
            pltpu.async_copy(x_sp.at[src_v[b]], vals_v[b], sem_g[b]).wait()
            s1 = pltpu.async_copy(vals_v[b], sum_sp.at[dst_v[b]],
                                  sem_s[b], add=True)
            s2 = pltpu.async_copy(ones_v, cnt_sp.at[dst_v[b]],
                                  sem_s[b], add=True)
            scat_descs[j] = (s1, s2)
            jf = j + 2
            if jf < NUM_CHUNKS:
                if jf - SLOTS in scat_descs:
                    for d in scat_descs.pop(jf - SLOTS):
                        d.wait()
                idx_descs[jf] = start_loads(jf)
        for descs in scat_descs.values():
            for d in descs:
                d.wait()

        plsc.subcore_barrier()

        out_base = cid * 2 * NP
        pltpu.sync_copy(sum_sp.at[pl.ds(off, PIECE)], piece_v)
        pltpu.sync_copy(piece_v, out_hbm.at[pl.ds(out_base + off, PIECE)])
        pltpu.sync_copy(cnt_sp.at[pl.ds(off, PIECE)], piece_v)
        pltpu.sync_copy(piece_v, out_hbm.at[pl.ds(out_base + NP + off, PIECE)])

    return sc_kernel(x_flat, zeros_np, ones_chunk, ei_flat)


def _tc_tail_body(p_ref, x_ref, wl_ref, bl_ref, wr_ref, o_ref):
    i = pl.program_id(0)
    p = p_ref[...]
    s = p[0:1, :] + p[2:3, :]
    c = p[1:2, :] + p[3:4, :]
    mean = s / jnp.maximum(c, 1.0)
    t = wl_ref[...] * mean + wr_ref[...] * x_ref[...]
    col = i * CB + jax.lax.broadcasted_iota(jnp.int32, (H, CB), 1)
    t = jnp.where(col < N, t, -jnp.inf)
    m = jnp.max(t, axis=1, keepdims=True)

    @pl.when(i == 0)
    def _init():
        o_ref[...] = m

    @pl.when(i > 0)
    def _acc():
        o_ref[...] = jnp.maximum(o_ref[...], m)

    @pl.when(i == GRID - 1)
    def _bias():
        o_ref[...] = o_ref[...] + bl_ref[...]


def _tc_tail(p4, xr, wlT, blT, wrT):
    col = pl.BlockSpec((H, 1), lambda i: (0, 0))
    out = pl.pallas_call(
        _tc_tail_body,
        grid=(GRID,),
        in_specs=[
            pl.BlockSpec((4, CB), lambda i: (0, i)),
            pl.BlockSpec((1, CB), lambda i: (0, i)),
            col, col, col,
        ],
        out_specs=pl.BlockSpec((H, 1), lambda i: (0, 0)),
        out_shape=jax.ShapeDtypeStruct((H, 1), jnp.float32),
    )(p4, xr, wlT, blT, wrT)
    return out.reshape(1, H)


def kernel(x, edge_index, batch, Wl, bl, Wr):
    del batch
    x_pad = jnp.concatenate([x.reshape(N), jnp.zeros((NP - N,), jnp.float32)])
    partials = _sc_segment_sums(
        x_pad,
        jnp.zeros((NP,), jnp.float32),
        jnp.ones((CHUNK,), jnp.float32),
        edge_index.reshape(-1),
    )
    return _tc_tail(
        partials.reshape(4, NP),
        x_pad.reshape(1, NP),
        Wl.reshape(H, 1),
        bl.reshape(H, 1),
        Wr.reshape(H, 1),
    )


# speedup vs baseline: 1.2905x; 1.2905x over previous
import functools

import jax
import jax.numpy as jnp
from jax import lax
from jax.experimental import pallas as pl
from jax.experimental.pallas import tpu as pltpu
from jax.experimental.pallas import tpu_sc as plsc

N = 100000
E = 6400000
H = 128

NP = 102400

NUM_CORES = 2
NUM_SUBCORES = 16
NUM_WORKERS = NUM_CORES * NUM_SUBCORES
EDGES_PER_WORKER = E // NUM_WORKERS
CHUNK = 25000
NUM_CHUNKS = EDGES_PER_WORKER // CHUNK
PIECE = NP // NUM_SUBCORES

CB = 6400
GRID = NP // CB


def _sc_segment_sums(x_flat, zeros_np, ones_chunk, ei_flat):
    mesh = plsc.VectorSubcoreMesh(core_axis_name="c", subcore_axis_name="s")

    @functools.partial(
        pl.kernel,
        out_type=jax.ShapeDtypeStruct((4 * NP,), jnp.float32),
        mesh=mesh,
        scratch_types=[
            pltpu.VMEM((CHUNK,), jnp.int32),
            pltpu.VMEM((CHUNK,), jnp.int32),
            pltpu.VMEM((CHUNK,), jnp.float32),
            pltpu.VMEM((CHUNK,), jnp.float32),
            pltpu.VMEM_SHARED((NP,), jnp.float32),
            pltpu.VMEM_SHARED((NP,), jnp.float32),
            pltpu.VMEM_SHARED((NP,), jnp.float32),
            pltpu.SemaphoreType.DMA,
            pltpu.SemaphoreType.DMA,
            pltpu.SemaphoreType.DMA,
        ],
    )
    def sc_kernel(x_hbm, zero_hbm, ones_hbm, ei_hbm, out_hbm,
                  src_v, dst_v, vals_v, ones_v, x_sp, sum_sp, cnt_sp,
                  sem_g, sem_s1, sem_s2):
        cid = lax.axis_index("c")
        sid = lax.axis_index("s")
        off = sid * PIECE
        piece = vals_v.at[pl.ds(0, PIECE)]

        pltpu.sync_copy(x_hbm.at[pl.ds(off, PIECE)], piece)
        pltpu.sync_copy(piece, x_sp.at[pl.ds(off, PIECE)])
        pltpu.sync_copy(zero_hbm.at[pl.ds(off, PIECE)], piece)
        pltpu.sync_copy(piece, sum_sp.at[pl.ds(off, PIECE)])
        pltpu.sync_copy(piece, cnt_sp.at[pl.ds(off, PIECE)])
        pltpu.sync_copy(ones_hbm, ones_v)

        plsc.subcore_barrier()

        wid = cid * NUM_SUBCORES + sid

        def chunk_body(j, carry):
            base = wid * EDGES_PER_WORKER + j * CHUNK
            pltpu.sync_copy(ei_hbm.at[pl.ds(base, CHUNK)], src_v)
            pltpu.sync_copy(ei_hbm.at[pl.ds(E + base, CHUNK)], dst_v)
            g = pltpu.async_copy(x_sp.at[src_v], vals_v, sem_g)
            s2 = pltpu.async_copy(ones_v, cnt_sp.at[dst_v], sem_s2, add=True)
            g.wait()
            s1 = pltpu.async_copy(vals_v, sum_sp.at[dst_v], sem_s1, add=True)
            s1.wait()
            s2.wait()
            return carry

        lax.fori_loop(0, NUM_CHUNKS, chunk_body, 0)

        plsc.subcore_barrier()

        out_base = cid * 2 * NP
        pltpu.sync_copy(sum_sp.at[pl.ds(off, PIECE)], piece)
        pltpu.sync_copy(piece, out_hbm.at[pl.ds(out_base + off, PIECE)])
        pltpu.sync_copy(cnt_sp.at[pl.ds(off, PIECE)], piece)
        pltpu.sync_copy(piece, out_hbm.at[pl.ds(out_base + NP + off, PIECE)])

    return sc_kernel(x_flat, zeros_np, ones_chunk, ei_flat)


def _tc_tail_body(p_ref, x_ref, wl_ref, bl_ref, wr_ref, o_ref):
    i = pl.program_id(0)
    p = p_ref[...]
    s = p[0:1, :] + p[2:3, :]
    c = p[1:2, :] + p[3:4, :]
    mean = s / jnp.maximum(c, 1.0)
    t = wl_ref[...] * mean + wr_ref[...] * x_ref[...]
    col = i * CB + jax.lax.broadcasted_iota(jnp.int32, (H, CB), 1)
    t = jnp.where(col < N, t, -jnp.inf)
    m = jnp.max(t, axis=1, keepdims=True)

    @pl.when(i == 0)
    def _init():
        o_ref[...] = m

    @pl.when(i > 0)
    def _acc():
        o_ref[...] = jnp.maximum(o_ref[...], m)

    @pl.when(i == GRID - 1)
    def _bias():
        o_ref[...] = o_ref[...] + bl_ref[...]


def _tc_tail(p4, xr, wlT, blT, wrT):
    col = pl.BlockSpec((H, 1), lambda i: (0, 0))
    out = pl.pallas_call(
        _tc_tail_body,
        grid=(GRID,),
        in_specs=[
            pl.BlockSpec((4, CB), lambda i: (0, i)),
            pl.BlockSpec((1, CB), lambda i: (0, i)),
            col, col, col,
        ],
        out_specs=pl.BlockSpec((H, 1), lambda i: (0, 0)),
        out_shape=jax.ShapeDtypeStruct((H, 1), jnp.float32),
    )(p4, xr, wlT, blT, wrT)
    return out.reshape(1, H)


def kernel(x, edge_index, batch, Wl, bl, Wr):
    del batch
    x_pad = jnp.concatenate([x.reshape(N), jnp.zeros((NP - N,), jnp.float32)])
    partials = _sc_segment_sums(
        x_pad,
        jnp.zeros((NP,), jnp.float32),
        jnp.ones((CHUNK,), jnp.float32),
        edge_index.reshape(-1),
    )
    return _tc_tail(
        partials.reshape(4, NP),
        x_pad.reshape(1, NP),
        Wl.reshape(H, 1),
        bl.reshape(H, 1),
        Wr.reshape(H, 1),
    )
